# Initial kernel scaffold; baseline (speedup 1.0000x reference)
import jax
import jax.numpy as jnp
from jax.experimental import pallas as pl


def kernel(x, edge_index, edge_attr, batch, params):
    raise NotImplementedError("write your pallas kernel here")


# speedup vs baseline: 1.0136x; 1.0136x over previous
import functools

import jax
import jax.numpy as jnp
from jax import lax
from jax.experimental import pallas as pl
from jax.experimental.pallas import tpu as pltpu
from jax.experimental.pallas import tpu_sc as plsc

NC, NS = 2, 16
NW = NC * NS
CHUNK = 128
DH = 32


def _leaky(v, s=0.01):
    return jnp.where(v >= 0, v, s * v)


def _dot(a, b):
    return jnp.dot(a, b, preferred_element_type=jnp.float32,
                   precision=jax.lax.Precision.HIGHEST)


def _sc_gather(table, idx3):
    nchunk = idx3.shape[1]
    e_per_w = nchunk * CHUNK
    E = NW * e_per_w
    dt = table.shape[1]
    mesh = plsc.VectorSubcoreMesh(core_axis_name="c", subcore_axis_name="s",
                                  num_cores=NC, num_subcores=NS)

    @functools.partial(
        pl.kernel, mesh=mesh,
        out_type=jax.ShapeDtypeStruct((E, dt), jnp.float32),
        scratch_types=[
            pltpu.VMEM((nchunk, CHUNK), jnp.int32),
            pltpu.VMEM((CHUNK, dt), jnp.float32),
            pltpu.SemaphoreType.DMA,
        ],
    )
    def k(table_hbm, idx_hbm, out_hbm, idx_v, buf, sem):
        w = lax.axis_index("c") * NS + lax.axis_index("s")
        pltpu.sync_copy(idx_hbm.at[w], idx_v)
        base = w * e_per_w

        def body(j, carry):
            pltpu.async_copy(table_hbm.at[idx_v.at[j]], buf, sem).wait()
            pltpu.sync_copy(buf, out_hbm.at[pl.ds(base + j * CHUNK, CHUNK)])
            return carry

        lax.fori_loop(0, nchunk, body, 0)

    return k(table, idx3)


def _sc_scatter_add(msg, idx3, zeros):
    n = zeros.shape[0]
    dt = msg.shape[1]
    nchunk = idx3.shape[1]
    e_per_w = nchunk * CHUNK
    rows_per = n // NS
    mesh = plsc.VectorSubcoreMesh(core_axis_name="c", subcore_axis_name="s",
                                  num_cores=NC, num_subcores=NS)

    @functools.partial(
        pl.kernel, mesh=mesh,
        out_type=jax.ShapeDtypeStruct((NC, n, dt), jnp.float32),
        scratch_types=[
            pltpu.VMEM((CHUNK,), jnp.int32),
            pltpu.VMEM((CHUNK, dt), jnp.float32),
            pltpu.VMEM_SHARED((n, dt), jnp.float32),
            pltpu.SemaphoreType.DMA,
        ],
    )
    def k(msg_hbm, idx_hbm, zeros_hbm, out_hbm, idx_v, buf, shared, sem):
        c = lax.axis_index("c")
        s = lax.axis_index("s")
        w = c * NS + s
        pltpu.sync_copy(zeros_hbm.at[pl.ds(s * rows_per, rows_per)],
                        shared.at[pl.ds(s * rows_per, rows_per)])
        plsc.subcore_barrier()
        base = w * e_per_w

        def body(j, carry):
            pltpu.sync_copy(idx_hbm.at[w, j], idx_v)
            pltpu.sync_copy(msg_hbm.at[pl.ds(base + j * CHUNK, CHUNK)], buf)
            pltpu.sync_copy(buf, shared.at[idx_v], add=True)
            return carry

        lax.fori_loop(0, nchunk, body, 0)
        plsc.subcore_barrier()
        pltpu.sync_copy(shared.at[pl.ds(s * rows_per, rows_per)],
                        out_hbm.at[c, pl.ds(s * rows_per, rows_per)])

    return k(msg, idx3, zeros)


def _tc_nfc(x, wnT, b):
    n, dn = x.shape
    tile = 2000

    def body(x_ref, w_ref, b_ref, o_ref):
        v = _leaky(
            _dot(x_ref[...], w_ref[...])
            + b_ref[...])
        o_ref[...] = jnp.concatenate(
            [v, jnp.zeros((tile, 128 - DH), jnp.float32)], axis=1)

    return pl.pallas_call(
        body,
        grid=(n // tile,),
        in_specs=[
            pl.BlockSpec((tile, dn), lambda i: (i, 0)),
            pl.BlockSpec((dn, DH), lambda i: (0, 0)),
            pl.BlockSpec((1, DH), lambda i: (0, 0)),
        ],
        out_specs=pl.BlockSpec((tile, 128), lambda i: (i, 0)),
        out_shape=jax.ShapeDtypeStruct((n, 128), jnp.float32),
    )(x, wnT, b)


def _tc_msg(hs, ea, waT, ba, wbT, b2):
    E, de = ea.shape
    tile = 1024

    def body(hs_ref, ea_ref, wa_ref, ba_ref, wb_ref, b2_ref, o_ref):
        ee = jnp.maximum(
            _dot(ea_ref[...], wa_ref[...])
            + ba_ref[...], 0.0)
        wflat = _dot(ee, wb_ref[...])
        hs = hs_ref[:, :DH]
        acc = _dot(hs, b2_ref[...])
        for i in range(DH):
            acc = acc + hs[:, i:i + 1] * wflat[:, i * DH:(i + 1) * DH]
        o_ref[...] = jnp.concatenate(
            [acc, jnp.zeros((tile, 128 - DH), jnp.float32)], axis=1)

    return pl.pallas_call(
        body,
        grid=(E // tile,),
        in_specs=[
            pl.BlockSpec((tile, 128), lambda i: (i, 0)),
            pl.BlockSpec((tile, de), lambda i: (i, 0)),
            pl.BlockSpec((de, DH), lambda i: (0, 0)),
            pl.BlockSpec((1, DH), lambda i: (0, 0)),
            pl.BlockSpec((DH, DH * DH), lambda i: (0, 0)),
            pl.BlockSpec((DH, DH), lambda i: (0, 0)),
        ],
        out_specs=pl.BlockSpec((tile, 128), lambda i: (i, 0)),
        out_shape=jax.ShapeDtypeStruct((E, 128), jnp.float32),
    )(hs, ea, waT, ba, wbT, b2)


def _tc_combine(p0, p1, h, root, bias, normalize):
    n = h.shape[0]
    tile = 2000
    dt = DH if normalize else 128

    def body(p0_ref, p1_ref, h_ref, r_ref, b_ref, o_ref):
        v = _leaky(
            p0_ref[...] + p1_ref[...]
            + _dot(h_ref[:, :DH], r_ref[...])
            + b_ref[...])
        if normalize:
            nrm = jnp.sqrt(jnp.sum(v * v, axis=1, keepdims=True))
            o_ref[...] = v / jnp.maximum(nrm, 1e-12)
        else:
            o_ref[...] = jnp.concatenate(
                [v, jnp.zeros((tile, 128 - DH), jnp.float32)], axis=1)

    return pl.pallas_call(
        body,
        grid=(n // tile,),
        in_specs=[
            pl.BlockSpec((tile, DH), lambda i: (i, 0)),
            pl.BlockSpec((tile, DH), lambda i: (i, 0)),
            pl.BlockSpec((tile, 128), lambda i: (i, 0)),
            pl.BlockSpec((DH, DH), lambda i: (0, 0)),
            pl.BlockSpec((1, DH), lambda i: (0, 0)),
        ],
        out_specs=pl.BlockSpec((tile, dt), lambda i: (i, 0)),
        out_shape=jax.ShapeDtypeStruct((n, dt), jnp.float32),
    )(p0, p1, h, root, bias)


def _tc_readout(hn, brow, bcol, G, num_timesteps,
                wlT, bl, wrT, br_, att, gbias, wihT, bih, whhT, bhh,
                fcoT, fcob, fcT, fcb):
    n = hn.shape[0]

    def body(hn_ref, brow_ref, bcol_ref,
             wl_ref, bl_ref, wr_ref, brr_ref, att_ref, gb_ref, wih_ref,
             bih_ref, whh_ref, bhh_ref, fco_ref, fcob_ref, fc_ref, fcb_ref,
             o_ref):
        hn = hn_ref[...]
        bm = brow_ref[...]
        gid = lax.broadcasted_iota(jnp.int32, (G, n), 0)
        M = (gid == bm).astype(jnp.float32)
        bc = bcol_ref[...]
        gid2 = lax.broadcasted_iota(jnp.int32, (n, G), 1)
        MT = (gid2 == bc).astype(jnp.float32)

        out = jnp.maximum(_dot(M, hn), 0.0)
        xl = _dot(hn, wl_ref[...]) + bl_ref[...]
        for _ in range(num_timesteps):
            xr = _dot(out, wr_ref[...]) + brr_ref[...]
            xrb = _dot(MT, xr)
            m = _leaky(xl + xrb)
            e = jnp.sum(m * att_ref[...], axis=1, keepdims=True)
            ex = jnp.exp(e)
            den = _dot(M, ex)
            denb = _dot(MT, den)
            alpha = ex / (denb + 1e-16)
            agg = _dot(M, alpha * xl) + gb_ref[...]
            hcell = jnp.where(agg > 0, agg, jnp.exp(agg) - 1.0)
            gi = _dot(hcell, wih_ref[...]) + bih_ref[...]
            gh = _dot(out, whh_ref[...]) + bhh_ref[...]
            r = jax.nn.sigmoid(gi[:, :DH] + gh[:, :DH])
            z = jax.nn.sigmoid(gi[:, DH:2 * DH] + gh[:, DH:2 * DH])
            ng = jnp.tanh(gi[:, 2 * DH:] + r * gh[:, 2 * DH:])
            out = jnp.maximum((1.0 - z) * ng + z * out, 0.0)
        out = _dot(out, fco_ref[...]) + fcob_ref[...]
        o_ref[...] = _dot(out, fc_ref[...]) + fcb_ref[...]

    return pl.pallas_call(
        body,
        out_shape=jax.ShapeDtypeStruct((G, fcT.shape[1]), jnp.float32),
    )(hn, brow, bcol, wlT, bl, wrT, br_, att, gbias,
      wihT, bih, whhT, bhh, fcoT, fcob, fcT, fcb)


def kernel(x, edge_index, edge_attr, batch, params):
    p = params
    n = x.shape[0]
    E = edge_attr.shape[0]
    G = 64
    num_timesteps = 4

    grain = NW * CHUNK
    e_pad = -(-E // grain) * grain
    n_pad = -(-(n + 1) // (NS * 8)) * (NS * 8)
    nchunk = e_pad // grain

    pad_e = e_pad - E
    src3 = jnp.concatenate(
        [edge_index[0], jnp.zeros((pad_e,), jnp.int32)]).reshape(
            NW, nchunk, CHUNK)
    dst3 = jnp.concatenate(
        [edge_index[1], jnp.full((pad_e,), n, jnp.int32)]).reshape(
            NW, nchunk, CHUNK)
    ea_pad = jnp.concatenate(
        [edge_attr, jnp.zeros((pad_e, edge_attr.shape[1]), jnp.float32)])
    zeros = jnp.zeros((n_pad, 128), jnp.float32)

    r2 = lambda v: v.reshape(1, -1)

    h = _tc_nfc(x, p['W_nfc'].T, r2(p['b_nfc']))

    for pre in ('gc1', 'gc2'):
        hs = _sc_gather(h, src3)
        msg = _tc_msg(hs, ea_pad, p[pre + '_Wa'].T, r2(p[pre + '_ba']),
                      p[pre + '_Wb'].T, p[pre + '_bb'].reshape(DH, DH))
        parts = _sc_scatter_add(msg, dst3, zeros)
        parts = parts[:, :n, :DH]
        h = _tc_combine(parts[0], parts[1], h, p[pre + '_root'],
                        r2(p[pre + '_bias']), normalize=(pre == 'gc2'))

    return _tc_readout(
        h, batch.reshape(1, n), batch.reshape(n, 1), G, num_timesteps,
        p['gat_Wl'].T, r2(p['gat_bl']), p['gat_Wr'].T, r2(p['gat_br']),
        r2(p['gat_att']), r2(p['gat_bias']),
        p['gru_Wih'].T, r2(p['gru_bih']), p['gru_Whh'].T, r2(p['gru_bhh']),
        p['fcout_W'].T, r2(p['fcout_b']), p['fc_W'].T, r2(p['fc_b']))


# speedup vs baseline: 2.1244x; 2.0958x over previous
import functools

import jax
import jax.numpy as jnp
from jax import lax
from jax.experimental import pallas as pl
from jax.experimental.pallas import tpu as pltpu
from jax.experimental.pallas import tpu_sc as plsc

NC, NS = 2, 16
NW = NC * NS
CHUNK = 128
DH = 32


def _leaky(v, s=0.01):
    return jnp.where(v >= 0, v, s * v)


def _dot(a, b):
    return jnp.dot(a, b, preferred_element_type=jnp.float32,
                   precision=jax.lax.Precision.HIGHEST)


def _dot_d(a, b):
    return jnp.dot(a, b, preferred_element_type=jnp.float32)


def _hi(a):
    return a.astype(jnp.bfloat16).astype(jnp.float32)


def _split3(a):
    h = _hi(a)
    return jnp.concatenate([h, a - h, h], axis=1)


def _rhs3(b):
    h = _hi(b)
    return jnp.concatenate([h, h, b - h], axis=0)


def _sc_gather(table, idx3):
    nchunk = idx3.shape[1]
    e_per_w = nchunk * CHUNK
    E = NW * e_per_w
    dt = table.shape[1]
    mesh = plsc.VectorSubcoreMesh(core_axis_name="c", subcore_axis_name="s",
                                  num_cores=NC, num_subcores=NS)

    @functools.partial(
        pl.kernel, mesh=mesh,
        out_type=jax.ShapeDtypeStruct((E, dt), jnp.float32),
        scratch_types=[
            pltpu.VMEM((nchunk, CHUNK), jnp.int32),
            pltpu.VMEM((CHUNK, dt), jnp.float32),
            pltpu.SemaphoreType.DMA,
        ],
    )
    def k(table_hbm, idx_hbm, out_hbm, idx_v, buf, sem):
        w = lax.axis_index("c") * NS + lax.axis_index("s")
        pltpu.sync_copy(idx_hbm.at[w], idx_v)
        base = w * e_per_w

        def body(j, carry):
            pltpu.async_copy(table_hbm.at[idx_v.at[j]], buf, sem).wait()
            pltpu.sync_copy(buf, out_hbm.at[pl.ds(base + j * CHUNK, CHUNK)])
            return carry

        lax.fori_loop(0, nchunk, body, 0)

    return k(table, idx3)


def _sc_scatter_add(msg, idx3, zeros):
    n = zeros.shape[0]
    dt = msg.shape[1]
    nchunk = idx3.shape[1]
    e_per_w = nchunk * CHUNK
    rows_per = n // NS
    mesh = plsc.VectorSubcoreMesh(core_axis_name="c", subcore_axis_name="s",
                                  num_cores=NC, num_subcores=NS)

    @functools.partial(
        pl.kernel, mesh=mesh,
        out_type=jax.ShapeDtypeStruct((NC, n, dt), jnp.float32),
        scratch_types=[
            pltpu.VMEM((CHUNK,), jnp.int32),
            pltpu.VMEM((CHUNK, dt), jnp.float32),
            pltpu.VMEM_SHARED((n, dt), jnp.float32),
            pltpu.SemaphoreType.DMA,
        ],
    )
    def k(msg_hbm, idx_hbm, zeros_hbm, out_hbm, idx_v, buf, shared, sem):
        c = lax.axis_index("c")
        s = lax.axis_index("s")
        w = c * NS + s
        pltpu.sync_copy(zeros_hbm.at[pl.ds(s * rows_per, rows_per)],
                        shared.at[pl.ds(s * rows_per, rows_per)])
        plsc.subcore_barrier()
        base = w * e_per_w

        def body(j, carry):
            pltpu.sync_copy(idx_hbm.at[w, j], idx_v)
            pltpu.sync_copy(msg_hbm.at[pl.ds(base + j * CHUNK, CHUNK)], buf)
            pltpu.sync_copy(buf, shared.at[idx_v], add=True)
            return carry

        lax.fori_loop(0, nchunk, body, 0)
        plsc.subcore_barrier()
        pltpu.sync_copy(shared.at[pl.ds(s * rows_per, rows_per)],
                        out_hbm.at[c, pl.ds(s * rows_per, rows_per)])

    return k(msg, idx3, zeros)


def _tc_nfc(x, wnT, b):
    n, dn = x.shape
    tile = 2000

    def body(x_ref, w_ref, b_ref, o_ref):
        v = _leaky(
            _dot(x_ref[...], w_ref[...])
            + b_ref[...])
        o_ref[...] = jnp.concatenate(
            [v, jnp.zeros((tile, 128 - DH), jnp.float32)], axis=1)

    return pl.pallas_call(
        body,
        grid=(n // tile,),
        in_specs=[
            pl.BlockSpec((tile, dn), lambda i: (i, 0)),
            pl.BlockSpec((dn, DH), lambda i: (0, 0)),
            pl.BlockSpec((1, DH), lambda i: (0, 0)),
        ],
        out_specs=pl.BlockSpec((tile, 128), lambda i: (i, 0)),
        out_shape=jax.ShapeDtypeStruct((n, 128), jnp.float32),
    )(x, wnT, b)


def _tc_msg(hs, ea, wa3, ba, w73, sum32, b23):
    E, de = ea.shape
    tile = 1024

    def body(hs_ref, ea_ref, wa_ref, ba_ref, w7_ref, s_ref, b2_ref, o_ref):
        ee = jnp.maximum(
            _dot_d(_split3(ea_ref[...]), wa_ref[...]) + ba_ref[...], 0.0)
        lhs96 = _split3(hs_ref[:, :DH])
        D = _dot_d(lhs96, w7_ref[...])
        ee128 = jnp.concatenate([ee, ee, ee, ee], axis=1)
        ee_tile = jnp.concatenate([ee128] * 8, axis=1)
        prod = ee_tile * D
        ph = _hi(prod)
        acc = (_dot_d(ph, s_ref[...]) + _dot_d(prod - ph, s_ref[...])
               + _dot_d(lhs96, b2_ref[...]))
        o_ref[...] = jnp.concatenate(
            [acc, jnp.zeros((tile, 128 - DH), jnp.float32)], axis=1)

    return pl.pallas_call(
        body,
        grid=(E // tile,),
        in_specs=[
            pl.BlockSpec((tile, 128), lambda i: (i, 0)),
            pl.BlockSpec((tile, de), lambda i: (i, 0)),
            pl.BlockSpec((3 * de, DH), lambda i: (0, 0)),
            pl.BlockSpec((1, DH), lambda i: (0, 0)),
            pl.BlockSpec((3 * DH, DH * DH), lambda i: (0, 0)),
            pl.BlockSpec((DH * DH, DH), lambda i: (0, 0)),
            pl.BlockSpec((3 * DH, DH), lambda i: (0, 0)),
        ],
        out_specs=pl.BlockSpec((tile, 128), lambda i: (i, 0)),
        out_shape=jax.ShapeDtypeStruct((E, 128), jnp.float32),
    )(hs, ea, wa3, ba, w73, sum32, b23)


def _tc_combine(p0, p1, h, root, bias, normalize):
    n = h.shape[0]
    tile = 2000
    dt = DH if normalize else 128

    def body(p0_ref, p1_ref, h_ref, r_ref, b_ref, o_ref):
        v = _leaky(
            p0_ref[...] + p1_ref[...]
            + _dot(h_ref[:, :DH], r_ref[...])
            + b_ref[...])
        if normalize:
            nrm = jnp.sqrt(jnp.sum(v * v, axis=1, keepdims=True))
            o_ref[...] = v / jnp.maximum(nrm, 1e-12)
        else:
            o_ref[...] = jnp.concatenate(
                [v, jnp.zeros((tile, 128 - DH), jnp.float32)], axis=1)

    return pl.pallas_call(
        body,
        grid=(n // tile,),
        in_specs=[
            pl.BlockSpec((tile, DH), lambda i: (i, 0)),
            pl.BlockSpec((tile, DH), lambda i: (i, 0)),
            pl.BlockSpec((tile, 128), lambda i: (i, 0)),
            pl.BlockSpec((DH, DH), lambda i: (0, 0)),
            pl.BlockSpec((1, DH), lambda i: (0, 0)),
        ],
        out_specs=pl.BlockSpec((tile, dt), lambda i: (i, 0)),
        out_shape=jax.ShapeDtypeStruct((n, dt), jnp.float32),
    )(p0, p1, h, root, bias)


def _tc_readout(hn, brow, bcol, G, num_timesteps,
                wlT, bl, wrT, br_, att, gbias, wihT, bih, whhT, bhh,
                fcoT, fcob, fcT, fcb):
    n = hn.shape[0]

    def body(hn_ref, brow_ref, bcol_ref,
             wl_ref, bl_ref, wr_ref, brr_ref, att_ref, gb_ref, wih_ref,
             bih_ref, whh_ref, bhh_ref, fco_ref, fcob_ref, fc_ref, fcb_ref,
             o_ref):
        hn = hn_ref[...]
        bm = brow_ref[...]
        gid = lax.broadcasted_iota(jnp.int32, (G, n), 0)
        M = (gid == bm).astype(jnp.float32)
        bc = bcol_ref[...]
        gid2 = lax.broadcasted_iota(jnp.int32, (n, G), 1)
        MT = (gid2 == bc).astype(jnp.float32)

        out = jnp.maximum(_dot(M, hn), 0.0)
        xl = _dot(hn, wl_ref[...]) + bl_ref[...]
        for _ in range(num_timesteps):
            xr = _dot(out, wr_ref[...]) + brr_ref[...]
            xrb = _dot(MT, xr)
            m = _leaky(xl + xrb)
            e = jnp.sum(m * att_ref[...], axis=1, keepdims=True)
            ex = jnp.exp(e)
            den = _dot(M, ex)
            denb = _dot(MT, den)
            alpha = ex / (denb + 1e-16)
            agg = _dot(M, alpha * xl) + gb_ref[...]
            hcell = jnp.where(agg > 0, agg, jnp.exp(agg) - 1.0)
            gi = _dot(hcell, wih_ref[...]) + bih_ref[...]
            gh = _dot(out, whh_ref[...]) + bhh_ref[...]
            r = jax.nn.sigmoid(gi[:, :DH] + gh[:, :DH])
            z = jax.nn.sigmoid(gi[:, DH:2 * DH] + gh[:, DH:2 * DH])
            ng = jnp.tanh(gi[:, 2 * DH:] + r * gh[:, 2 * DH:])
            out = jnp.maximum((1.0 - z) * ng + z * out, 0.0)
        out = _dot(out, fco_ref[...]) + fcob_ref[...]
        o_ref[...] = _dot(out, fc_ref[...]) + fcb_ref[...]

    return pl.pallas_call(
        body,
        out_shape=jax.ShapeDtypeStruct((G, fcT.shape[1]), jnp.float32),
    )(hn, brow, bcol, wlT, bl, wrT, br_, att, gbias,
      wihT, bih, whhT, bhh, fcoT, fcob, fcT, fcb)


def kernel(x, edge_index, edge_attr, batch, params):
    p = params
    n = x.shape[0]
    E = edge_attr.shape[0]
    G = 64
    num_timesteps = 4

    grain = NW * CHUNK
    e_pad = -(-E // grain) * grain
    n_pad = -(-(n + 1) // (NS * 8)) * (NS * 8)
    nchunk = e_pad // grain

    pad_e = e_pad - E
    src3 = jnp.concatenate(
        [edge_index[0], jnp.zeros((pad_e,), jnp.int32)]).reshape(
            NW, nchunk, CHUNK)
    dst3 = jnp.concatenate(
        [edge_index[1], jnp.full((pad_e,), n, jnp.int32)]).reshape(
            NW, nchunk, CHUNK)
    ea_pad = jnp.concatenate(
        [edge_attr, jnp.zeros((pad_e, edge_attr.shape[1]), jnp.float32)])
    zeros = jnp.zeros((n_pad, 128), jnp.float32)

    r2 = lambda v: v.reshape(1, -1)
    sum32 = (jnp.arange(DH * DH)[:, None] // DH
             == jnp.arange(DH)[None, :]).astype(jnp.float32)

    h = _tc_nfc(x, p['W_nfc'].T, r2(p['b_nfc']))

    for pre in ('gc1', 'gc2'):
        hs = _sc_gather(h, src3)
        msg = _tc_msg(hs, ea_pad, _rhs3(p[pre + '_Wa'].T), r2(p[pre + '_ba']),
                      _rhs3(p[pre + '_Wb'].reshape(DH, DH * DH)), sum32,
                      _rhs3(p[pre + '_bb'].reshape(DH, DH)))
        parts = _sc_scatter_add(msg, dst3, zeros)
        parts = parts[:, :n, :DH]
        h = _tc_combine(parts[0], parts[1], h, p[pre + '_root'],
                        r2(p[pre + '_bias']), normalize=(pre == 'gc2'))

    return _tc_readout(
        h, batch.reshape(1, n), batch.reshape(n, 1), G, num_timesteps,
        p['gat_Wl'].T, r2(p['gat_bl']), p['gat_Wr'].T, r2(p['gat_br']),
        r2(p['gat_att']), r2(p['gat_bias']),
        p['gru_Wih'].T, r2(p['gru_bih']), p['gru_Whh'].T, r2(p['gru_bhh']),
        p['fcout_W'].T, r2(p['fcout_b']), p['fc_W'].T, r2(p['fc_b']))


# speedup vs baseline: 2.3573x; 1.1096x over previous
import functools

import jax
import jax.numpy as jnp
from jax import lax
from jax.experimental import pallas as pl
from jax.experimental.pallas import tpu as pltpu
from jax.experimental.pallas import tpu_sc as plsc

NC, NS = 2, 16
NW = NC * NS
CHUNK = 128
DH = 32


def _leaky(v, s=0.01):
    return jnp.where(v >= 0, v, s * v)


def _dot(a, b):
    return jnp.dot(a, b, preferred_element_type=jnp.float32,
                   precision=jax.lax.Precision.HIGHEST)


def _dot_d(a, b):
    return jnp.dot(a, b, preferred_element_type=jnp.float32)


def _hi(a):
    return a.astype(jnp.bfloat16).astype(jnp.float32)


def _split3(a):
    h = _hi(a)
    return jnp.concatenate([h, a - h, h], axis=1)


def _rhs3(b):
    h = _hi(b)
    return jnp.concatenate([h, h, b - h], axis=0)


def _sc_gather(table, idx3):
    nchunk = idx3.shape[1]
    e_per_w = nchunk * CHUNK
    E = NW * e_per_w
    dt = table.shape[1]
    mesh = plsc.VectorSubcoreMesh(core_axis_name="c", subcore_axis_name="s",
                                  num_cores=NC, num_subcores=NS)

    @functools.partial(
        pl.kernel, mesh=mesh,
        out_type=jax.ShapeDtypeStruct((E, dt), jnp.float32),
        scratch_types=[
            pltpu.VMEM((nchunk, CHUNK), jnp.int32),
            pltpu.VMEM((CHUNK, dt), jnp.float32),
            pltpu.VMEM((CHUNK, dt), jnp.float32),
            pltpu.SemaphoreType.DMA,
            pltpu.SemaphoreType.DMA,
        ],
    )
    def k(table_hbm, idx_hbm, out_hbm, idx_v, buf0, buf1, sem0, sem1):
        w = lax.axis_index("c") * NS + lax.axis_index("s")
        pltpu.sync_copy(idx_hbm.at[w], idx_v)
        base = w * e_per_w
        bufs, sems = (buf0, buf1), (sem0, sem1)
        pltpu.async_copy(table_hbm.at[idx_v.at[0]], buf0, sem0)

        def body(j2, carry):
            for ph in range(2):
                j = 2 * j2 + ph
                b, s = bufs[ph], sems[ph]
                nb, ns_ = bufs[1 - ph], sems[1 - ph]

                @pl.when(j + 1 < nchunk)
                def _():
                    pltpu.async_copy(table_hbm.at[idx_v.at[j + 1]], nb, ns_)

                pltpu.make_async_copy(table_hbm.at[idx_v.at[j]], b, s).wait()
                pltpu.sync_copy(b, out_hbm.at[pl.ds(base + j * CHUNK, CHUNK)])
            return carry

        lax.fori_loop(0, nchunk // 2, body, 0)

    return k(table, idx3)


def _sc_scatter_add(msg, idx3, zeros):
    n = zeros.shape[0]
    dt = msg.shape[1]
    nchunk = idx3.shape[1]
    e_per_w = nchunk * CHUNK
    rows_per = n // NS
    mesh = plsc.VectorSubcoreMesh(core_axis_name="c", subcore_axis_name="s",
                                  num_cores=NC, num_subcores=NS)

    @functools.partial(
        pl.kernel, mesh=mesh,
        out_type=jax.ShapeDtypeStruct((NC, n, dt), jnp.float32),
        scratch_types=[
            pltpu.VMEM((CHUNK,), jnp.int32),
            pltpu.VMEM((CHUNK,), jnp.int32),
            pltpu.VMEM((CHUNK, dt), jnp.float32),
            pltpu.VMEM((CHUNK, dt), jnp.float32),
            pltpu.VMEM_SHARED((n, dt), jnp.float32),
            pltpu.SemaphoreType.DMA,
            pltpu.SemaphoreType.DMA,
            pltpu.SemaphoreType.DMA,
            pltpu.SemaphoreType.DMA,
        ],
    )
    def k(msg_hbm, idx_hbm, zeros_hbm, out_hbm, idx0, idx1, buf0, buf1,
          shared, semi0, semi1, semm0, semm1):
        c = lax.axis_index("c")
        s = lax.axis_index("s")
        w = c * NS + s
        pltpu.sync_copy(zeros_hbm.at[pl.ds(s * rows_per, rows_per)],
                        shared.at[pl.ds(s * rows_per, rows_per)])
        plsc.subcore_barrier()
        base = w * e_per_w
        idxs, bufs = (idx0, idx1), (buf0, buf1)
        semis, semms = (semi0, semi1), (semm0, semm1)

        def start(j, ph):
            pltpu.async_copy(idx_hbm.at[w, j], idxs[ph], semis[ph])
            pltpu.async_copy(msg_hbm.at[pl.ds(base + j * CHUNK, CHUNK)],
                             bufs[ph], semms[ph])

        start(0, 0)

        def body(j2, carry):
            for ph in range(2):
                j = 2 * j2 + ph

                @pl.when(j + 1 < nchunk)
                def _():
                    start(j + 1, 1 - ph)

                pltpu.make_async_copy(idx_hbm.at[w, j], idxs[ph],
                                      semis[ph]).wait()
                pltpu.make_async_copy(msg_hbm.at[pl.ds(base, CHUNK)],
                                      bufs[ph], semms[ph]).wait()
                pltpu.sync_copy(bufs[ph], shared.at[idxs[ph]], add=True)
            return carry

        lax.fori_loop(0, nchunk // 2, body, 0)
        plsc.subcore_barrier()
        pltpu.sync_copy(shared.at[pl.ds(s * rows_per, rows_per)],
                        out_hbm.at[c, pl.ds(s * rows_per, rows_per)])

    return k(msg, idx3, zeros)


def _tc_nfc(x, wnT, b):
    n, dn = x.shape
    tile = 2000

    def body(x_ref, w_ref, b_ref, o_ref):
        v = _leaky(
            _dot(x_ref[...], w_ref[...])
            + b_ref[...])
        o_ref[...] = jnp.concatenate(
            [v, jnp.zeros((tile, 128 - DH), jnp.float32)], axis=1)

    return pl.pallas_call(
        body,
        grid=(n // tile,),
        in_specs=[
            pl.BlockSpec((tile, dn), lambda i: (i, 0)),
            pl.BlockSpec((dn, DH), lambda i: (0, 0)),
            pl.BlockSpec((1, DH), lambda i: (0, 0)),
        ],
        out_specs=pl.BlockSpec((tile, 128), lambda i: (i, 0)),
        out_shape=jax.ShapeDtypeStruct((n, 128), jnp.float32),
    )(x, wnT, b)


def _tc_msg(hs, ea, wa3, ba, w73, sum32, b23):
    E, de = ea.shape
    tile = 1024

    def body(hs_ref, ea_ref, wa_ref, ba_ref, w7_ref, s_ref, b2_ref, o_ref):
        ee = jnp.maximum(
            _dot_d(_split3(ea_ref[...]), wa_ref[...]) + ba_ref[...], 0.0)
        lhs96 = _split3(hs_ref[:, :DH])
        D = _dot_d(lhs96, w7_ref[...])
        ee128 = jnp.concatenate([ee, ee, ee, ee], axis=1)
        ee_tile = jnp.concatenate([ee128] * 8, axis=1)
        prod = ee_tile * D
        ph = _hi(prod)
        acc = (_dot_d(ph, s_ref[...]) + _dot_d(prod - ph, s_ref[...])
               + _dot_d(lhs96, b2_ref[...]))
        o_ref[...] = jnp.concatenate(
            [acc, jnp.zeros((tile, 128 - DH), jnp.float32)], axis=1)

    return pl.pallas_call(
        body,
        grid=(E // tile,),
        in_specs=[
            pl.BlockSpec((tile, 128), lambda i: (i, 0)),
            pl.BlockSpec((tile, de), lambda i: (i, 0)),
            pl.BlockSpec((3 * de, DH), lambda i: (0, 0)),
            pl.BlockSpec((1, DH), lambda i: (0, 0)),
            pl.BlockSpec((3 * DH, DH * DH), lambda i: (0, 0)),
            pl.BlockSpec((DH * DH, DH), lambda i: (0, 0)),
            pl.BlockSpec((3 * DH, DH), lambda i: (0, 0)),
        ],
        out_specs=pl.BlockSpec((tile, 128), lambda i: (i, 0)),
        out_shape=jax.ShapeDtypeStruct((E, 128), jnp.float32),
    )(hs, ea, wa3, ba, w73, sum32, b23)


def _tc_combine(p0, p1, h, root, bias, normalize):
    n = h.shape[0]
    tile = 2000
    dt = DH if normalize else 128

    def body(p0_ref, p1_ref, h_ref, r_ref, b_ref, o_ref):
        v = _leaky(
            p0_ref[...] + p1_ref[...]
            + _dot(h_ref[:, :DH], r_ref[...])
            + b_ref[...])
        if normalize:
            nrm = jnp.sqrt(jnp.sum(v * v, axis=1, keepdims=True))
            o_ref[...] = v / jnp.maximum(nrm, 1e-12)
        else:
            o_ref[...] = jnp.concatenate(
                [v, jnp.zeros((tile, 128 - DH), jnp.float32)], axis=1)

    return pl.pallas_call(
        body,
        grid=(n // tile,),
        in_specs=[
            pl.BlockSpec((tile, DH), lambda i: (i, 0)),
            pl.BlockSpec((tile, DH), lambda i: (i, 0)),
            pl.BlockSpec((tile, 128), lambda i: (i, 0)),
            pl.BlockSpec((DH, DH), lambda i: (0, 0)),
            pl.BlockSpec((1, DH), lambda i: (0, 0)),
        ],
        out_specs=pl.BlockSpec((tile, dt), lambda i: (i, 0)),
        out_shape=jax.ShapeDtypeStruct((n, dt), jnp.float32),
    )(p0, p1, h, root, bias)


def _tc_readout(hn, brow, bcol, G, num_timesteps,
                wlT, bl, wrT, br_, att, gbias, wihT, bih, whhT, bhh,
                fcoT, fcob, fcT, fcb):
    n = hn.shape[0]

    def body(hn_ref, brow_ref, bcol_ref,
             wl_ref, bl_ref, wr_ref, brr_ref, att_ref, gb_ref, wih_ref,
             bih_ref, whh_ref, bhh_ref, fco_ref, fcob_ref, fc_ref, fcb_ref,
             o_ref):
        hn = hn_ref[...]
        bm = brow_ref[...]
        gid = lax.broadcasted_iota(jnp.int32, (G, n), 0)
        M = (gid == bm).astype(jnp.float32)
        bc = bcol_ref[...]
        gid2 = lax.broadcasted_iota(jnp.int32, (n, G), 1)
        MT = (gid2 == bc).astype(jnp.float32)

        out = jnp.maximum(_dot(M, hn), 0.0)
        xl = _dot(hn, wl_ref[...]) + bl_ref[...]
        for _ in range(num_timesteps):
            xr = _dot(out, wr_ref[...]) + brr_ref[...]
            xrb = _dot(MT, xr)
            m = _leaky(xl + xrb)
            e = jnp.sum(m * att_ref[...], axis=1, keepdims=True)
            ex = jnp.exp(e)
            den = _dot(M, ex)
            denb = _dot(MT, den)
            alpha = ex / (denb + 1e-16)
            agg = _dot(M, alpha * xl) + gb_ref[...]
            hcell = jnp.where(agg > 0, agg, jnp.exp(agg) - 1.0)
            gi = _dot(hcell, wih_ref[...]) + bih_ref[...]
            gh = _dot(out, whh_ref[...]) + bhh_ref[...]
            r = jax.nn.sigmoid(gi[:, :DH] + gh[:, :DH])
            z = jax.nn.sigmoid(gi[:, DH:2 * DH] + gh[:, DH:2 * DH])
            ng = jnp.tanh(gi[:, 2 * DH:] + r * gh[:, 2 * DH:])
            out = jnp.maximum((1.0 - z) * ng + z * out, 0.0)
        out = _dot(out, fco_ref[...]) + fcob_ref[...]
        o_ref[...] = _dot(out, fc_ref[...]) + fcb_ref[...]

    return pl.pallas_call(
        body,
        out_shape=jax.ShapeDtypeStruct((G, fcT.shape[1]), jnp.float32),
    )(hn, brow, bcol, wlT, bl, wrT, br_, att, gbias,
      wihT, bih, whhT, bhh, fcoT, fcob, fcT, fcb)


def kernel(x, edge_index, edge_attr, batch, params):
    p = params
    n = x.shape[0]
    E = edge_attr.shape[0]
    G = 64
    num_timesteps = 4

    grain = NW * CHUNK
    e_pad = -(-E // grain) * grain
    n_pad = -(-(n + 1) // (NS * 8)) * (NS * 8)
    nchunk = e_pad // grain

    pad_e = e_pad - E
    src3 = jnp.concatenate(
        [edge_index[0], jnp.zeros((pad_e,), jnp.int32)]).reshape(
            NW, nchunk, CHUNK)
    dst3 = jnp.concatenate(
        [edge_index[1], jnp.full((pad_e,), n, jnp.int32)]).reshape(
            NW, nchunk, CHUNK)
    ea_pad = jnp.concatenate(
        [edge_attr, jnp.zeros((pad_e, edge_attr.shape[1]), jnp.float32)])
    zeros = jnp.zeros((n_pad, 128), jnp.float32)

    r2 = lambda v: v.reshape(1, -1)
    sum32 = (jnp.arange(DH * DH)[:, None] // DH
             == jnp.arange(DH)[None, :]).astype(jnp.float32)

    h = _tc_nfc(x, p['W_nfc'].T, r2(p['b_nfc']))

    for pre in ('gc1', 'gc2'):
        hs = _sc_gather(h, src3)
        msg = _tc_msg(hs, ea_pad, _rhs3(p[pre + '_Wa'].T), r2(p[pre + '_ba']),
                      _rhs3(p[pre + '_Wb'].reshape(DH, DH * DH)), sum32,
                      _rhs3(p[pre + '_bb'].reshape(DH, DH)))
        parts = _sc_scatter_add(msg, dst3, zeros)
        parts = parts[:, :n, :DH]
        h = _tc_combine(parts[0], parts[1], h, p[pre + '_root'],
                        r2(p[pre + '_bias']), normalize=(pre == 'gc2'))

    return _tc_readout(
        h, batch.reshape(1, n), batch.reshape(n, 1), G, num_timesteps,
        p['gat_Wl'].T, r2(p['gat_bl']), p['gat_Wr'].T, r2(p['gat_br']),
        r2(p['gat_att']), r2(p['gat_bias']),
        p['gru_Wih'].T, r2(p['gru_bih']), p['gru_Whh'].T, r2(p['gru_bhh']),
        p['fcout_W'].T, r2(p['fcout_b']), p['fc_W'].T, r2(p['fc_b']))
